# Initial kernel scaffold; baseline (speedup 1.0000x reference)
import jax
import jax.numpy as jnp
from jax.experimental import pallas as pl


def kernel(surface_pcl, query_pcl, params):
    raise NotImplementedError("write your pallas kernel here")


# speedup vs baseline: 3.7349x; 3.7349x over previous
import functools
import math

import jax
import jax.numpy as jnp
from jax.experimental import pallas as pl
from jax.experimental.pallas import tpu as pltpu

_CP = pltpu.CompilerParams(vmem_limit_bytes=128 * 1024 * 1024)

_B, _N, _NQ = 4, 8192, 8192
_SA = [(1024, 0.1, 16, (6, 16, 32)), (512, 0.2, 32, (35, 32, 64)),
       (128, 0.4, 64, (67, 64, 128))]
_FP = [(35, 32, 32), (67, 32, 32), (131, 32, 32)]
_BN_SCALE = 1.0 / math.sqrt(1.0 + 1e-5)


def _fold(layer):
    w = layer['W'] * (layer['g'] * _BN_SCALE)[:, None]
    return w, layer['b'][None, :]


def _fps_body(n, npoint, xyz_t_ref, nxyz_ref):
    iota_n = jax.lax.broadcasted_iota(jnp.int32, (1, n), 1)
    iota_p = jax.lax.broadcasted_iota(jnp.int32, (1, npoint), 1)
    x = xyz_t_ref[...]

    def step(i, carry):
        dists, c, nxyz = carry
        dx = x[0:1, :] - c[0:1, :]
        dy = x[1:2, :] - c[1:2, :]
        dz = x[2:3, :] - c[2:3, :]
        d = dx * dx + dy * dy + dz * dz
        dists = jnp.minimum(dists, d)
        m = jnp.max(dists)
        nxt = jnp.min(jnp.where(dists == m, iota_n, n))
        c = jnp.sum(jnp.where(iota_n == nxt, x, 0.0), axis=1,
                    keepdims=True)
        nxyz = jnp.where(iota_p == i, c, nxyz)
        return dists, c, nxyz

    d0 = jnp.full((1, n), 1e10, jnp.float32)
    c0 = x[:, 0:1]
    nxyz0 = jnp.where(iota_p == 0, c0, jnp.zeros((3, npoint), jnp.float32))
    _, _, nxyz = jax.lax.fori_loop(1, npoint, step, (d0, c0, nxyz0))
    nxyz_ref[...] = nxyz


def _fps(xyz_t, npoint):
    b, _, n = xyz_t.shape
    return pl.pallas_call(
        functools.partial(_fps_body, n, npoint),
        grid=(b,),
        in_specs=[pl.BlockSpec((None, 3, n), lambda i: (i, 0, 0))],
        out_specs=pl.BlockSpec((None, 3, npoint), lambda i: (i, 0, 0)),
        out_shape=jax.ShapeDtypeStruct((b, 3, npoint), jnp.float32),
    )(xyz_t)


def _sqd(a_ref, bt_ref):
    qx, qy, qz = a_ref[:, 0:1], a_ref[:, 1:2], a_ref[:, 2:3]
    px, py, pz = bt_ref[0:1, :], bt_ref[1:2, :], bt_ref[2:3, :]
    aa = (qx * qx + qy * qy) + qz * qz
    bb = (px * px + py * py) + pz * pz
    ab = (qx * px + qy * py) + qz * pz
    return jnp.maximum(aa + bb - 2.0 * ab, 0.0)


def _sa_body(n, k, r2, xyz_t_ref, xyz_ref, feats_ref, nq_ref,
             wa_ref, wb_ref, b1_ref, w2_ref, b2_ref, out_ref):
    q = nq_ref.shape[0]
    p = (jnp.dot(xyz_ref[...], wa_ref[...].T,
                 preferred_element_type=jnp.float32,
                 precision=jax.lax.Precision.HIGHEST) +
         jnp.dot(feats_ref[...], wb_ref[...].T,
                 preferred_element_type=jnp.float32,
                 precision=jax.lax.Precision.HIGHEST))
    off = b1_ref[...] - jnp.dot(nq_ref[...], wa_ref[...].T,
                                preferred_element_type=jnp.float32,
                 precision=jax.lax.Precision.HIGHEST)
    d = _sqd(nq_ref, xyz_t_ref)
    iota = jax.lax.broadcasted_iota(jnp.int32, (q, n), 1)
    keyed = jnp.where(d <= r2, iota, n)
    pooled = jnp.full((q, out_ref.shape[1]), -jnp.inf, jnp.float32)
    for _ in range(k):
        m = jnp.min(keyed, axis=1, keepdims=True)
        valid = m < n
        oh = (keyed == m) & valid
        pg = jnp.dot(oh.astype(jnp.float32), p,
                     preferred_element_type=jnp.float32,
                 precision=jax.lax.Precision.HIGHEST)
        h1 = jnp.maximum(pg + off, 0.0)
        h2 = jnp.maximum(jnp.dot(h1, w2_ref[...].T,
                                 preferred_element_type=jnp.float32,
                 precision=jax.lax.Precision.HIGHEST)
                         + b2_ref[...], 0.0)
        pooled = jnp.where(valid, jnp.maximum(pooled, h2), pooled)
        keyed = jnp.where(oh, n, keyed)
    out_ref[...] = pooled


def _sa_stage(xyz_t, xyz, feats, new_xyz, layers, radius, k, qblk):
    b, n, c = feats.shape
    s = new_xyz.shape[1]
    (w1, b1), (w2, b2) = _fold(layers[0]), _fold(layers[1])
    wa, wb = w1[:, :3], w1[:, 3:]
    c1, c2 = w1.shape[0], w2.shape[0]
    grid = (b, s // qblk)
    return pl.pallas_call(
        functools.partial(_sa_body, n, k, float(radius) * float(radius)),
        grid=grid,
        in_specs=[
            pl.BlockSpec((None, 3, n), lambda i, j: (i, 0, 0)),
            pl.BlockSpec((None, n, 3), lambda i, j: (i, 0, 0)),
            pl.BlockSpec((None, n, c), lambda i, j: (i, 0, 0)),
            pl.BlockSpec((None, qblk, 3), lambda i, j: (i, j, 0)),
            pl.BlockSpec((c1, 3), lambda i, j: (0, 0)),
            pl.BlockSpec((c1, c), lambda i, j: (0, 0)),
            pl.BlockSpec((1, c1), lambda i, j: (0, 0)),
            pl.BlockSpec((c2, c1), lambda i, j: (0, 0)),
            pl.BlockSpec((1, c2), lambda i, j: (0, 0)),
        ],
        out_specs=pl.BlockSpec((None, qblk, c2), lambda i, j: (i, j, 0)),
        out_shape=jax.ShapeDtypeStruct((b, s, c2), jnp.float32),
        compiler_params=_CP,
    )(xyz_t, xyz, feats, new_xyz, wa, wb, b1, w2, b2)


def _fp_body(npoint, q_ref, kx_t_ref, kf_ref, wa_ref, wb_ref, b1_ref,
             w2_ref, b2_ref, out_ref):
    qb = q_ref.shape[0]
    d = _sqd(q_ref, kx_t_ref)
    iota = jax.lax.broadcasted_iota(jnp.int32, (qb, npoint), 1)
    acc = jnp.zeros((qb, kf_ref.shape[1]), jnp.float32)
    wsum = jnp.zeros((qb, 1), jnp.float32)
    for _ in range(3):
        m = jnp.min(d, axis=1, keepdims=True)
        sel = jnp.min(jnp.where(d == m, iota, npoint), axis=1,
                      keepdims=True)
        oh = iota == sel
        w = 1.0 / (m + 1e-8)
        acc += w * jnp.dot(oh.astype(jnp.float32), kf_ref[...],
                           preferred_element_type=jnp.float32,
                 precision=jax.lax.Precision.HIGHEST)
        wsum += w
        d = jnp.where(oh, jnp.inf, d)
    interp = acc / wsum
    h1 = jnp.maximum(
        jnp.dot(interp, wa_ref[...].T, preferred_element_type=jnp.float32,
                 precision=jax.lax.Precision.HIGHEST)
        + jnp.dot(q_ref[...], wb_ref[...].T,
                  preferred_element_type=jnp.float32,
                 precision=jax.lax.Precision.HIGHEST)
        + b1_ref[...], 0.0)
    h2 = jnp.maximum(jnp.dot(h1, w2_ref[...].T,
                             preferred_element_type=jnp.float32,
                 precision=jax.lax.Precision.HIGHEST)
                     + b2_ref[...], 0.0)
    out_ref[...] = h2


def _fp_stage(q, kx_t, kf, layers, qblk):
    b, nq, _ = q.shape
    npoint, c = kf.shape[1], kf.shape[2]
    (w1, b1), (w2, b2) = _fold(layers[0]), _fold(layers[1])
    wa, wb = w1[:, :c], w1[:, c:]
    c1, c2 = w1.shape[0], w2.shape[0]
    return pl.pallas_call(
        functools.partial(_fp_body, npoint),
        grid=(b, nq // qblk),
        in_specs=[
            pl.BlockSpec((None, qblk, 3), lambda i, j: (i, j, 0)),
            pl.BlockSpec((None, 3, npoint), lambda i, j: (i, 0, 0)),
            pl.BlockSpec((None, npoint, c), lambda i, j: (i, 0, 0)),
            pl.BlockSpec((c1, c), lambda i, j: (0, 0)),
            pl.BlockSpec((c1, 3), lambda i, j: (0, 0)),
            pl.BlockSpec((1, c1), lambda i, j: (0, 0)),
            pl.BlockSpec((c2, c1), lambda i, j: (0, 0)),
            pl.BlockSpec((1, c2), lambda i, j: (0, 0)),
        ],
        out_specs=pl.BlockSpec((None, qblk, c2), lambda i, j: (i, j, 0)),
        out_shape=jax.ShapeDtypeStruct((b, nq, c2), jnp.float32),
        compiler_params=_CP,
    )(q, kx_t, kf, wa, wb, b1, w2, b2)


def kernel(surface_pcl, query_pcl, params):
    xyz = surface_pcl
    xyz_t = jnp.swapaxes(xyz, 1, 2)
    feats = xyz
    q = jnp.swapaxes(query_pcl, 1, 2)

    sa_qblk = [64, 128, 128]
    outs = []
    for i in range(3):
        npoint, radius, k, _ = _SA[i]
        nx_t = _fps(xyz_t, npoint)
        nx = jnp.swapaxes(nx_t, 1, 2)
        feats = _sa_stage(xyz_t, xyz, feats, nx, params['sa'][i],
                          radius, k, min(sa_qblk[i], npoint))
        xyz, xyz_t = nx, nx_t
        o = _fp_stage(q, nx_t, feats, params['fp'][i],
                      min(512, q.shape[1]))
        outs.append(jnp.swapaxes(o, 1, 2))
    return jnp.concatenate(outs, axis=1)
